# baseline (speedup 1.0000x reference)
import functools

import jax
import jax.numpy as jnp
from jax import lax
from jax.experimental import pallas as pl
from jax.experimental.pallas import tpu as pltpu
from jax.experimental.pallas import tpu_sc as plsc

N = 10000
E = 320000
DF = 128
DE = 16
L = 32

NC = 2
NS = 16
NW = NC * NS
EW = E // NW
CH = 80
NCH = EW // CH
NP = 10240
NROWS = NP // NS

E4 = E // 4


def _make_sc_block(write_e: bool):
    mesh = plsc.VectorSubcoreMesh(
        core_axis_name="c", subcore_axis_name="s", num_cores=NC, num_subcores=NS
    )
    out_type = []
    if write_e:
        out_type.append(jax.ShapeDtypeStruct((NW, EW, L), jnp.float32))
    out_type.append(jax.ShapeDtypeStruct((NC, NP, L), jnp.float32))

    scratch = [
        pltpu.VMEM((NCH, CH), jnp.int32),
        pltpu.VMEM((NCH, CH), jnp.int32),
        pltpu.VMEM((2, CH, L), jnp.float32),
        pltpu.VMEM((2, CH, L), jnp.float32),
        pltpu.VMEM((2, CH, L), jnp.float32),
        pltpu.VMEM((2, CH, L), jnp.float32),
        pltpu.SemaphoreType.DMA,
        pltpu.SemaphoreType.DMA,
        pltpu.SemaphoreType.DMA,
        pltpu.SemaphoreType.DMA,
        pltpu.VMEM_SHARED((NP, L), jnp.float32),
    ]

    def body(s_hbm, r_hbm, base_hbm, S_hbm, R_hbm, z_hbm, *rest):
        if write_e:
            e_out, agg_out = rest[0], rest[1]
            scr = rest[2:]
        else:
            agg_out = rest[0]
            scr = rest[1:]
        s_v, r_v, b_v, sr_v, rr_v, e_v, sem_b, sem_g, sem_sl, sem_si, agg_sh = scr

        c = lax.axis_index("c")
        sid = lax.axis_index("s")
        t = c * NS + sid

        pltpu.sync_copy(
            z_hbm.at[pl.ds(sid * NROWS, NROWS)],
            agg_sh.at[pl.ds(sid * NROWS, NROWS)],
        )
        pltpu.sync_copy(s_hbm.at[t], s_v)
        pltpu.sync_copy(r_hbm.at[t], r_v)
        plsc.subcore_barrier()

        def issue_in(j, b):
            pltpu.async_copy(base_hbm.at[t, pl.ds(j * CH, CH)], b_v.at[b], sem_b)
            pltpu.async_copy(S_hbm.at[s_v.at[j]], sr_v.at[b], sem_g)
            pltpu.async_copy(R_hbm.at[r_v.at[j]], rr_v.at[b], sem_g)

        def wait_in(j, b):
            pltpu.make_async_copy(
                base_hbm.at[t, pl.ds(j * CH, CH)], b_v.at[b], sem_b).wait()
            pltpu.make_async_copy(S_hbm.at[s_v.at[j]], sr_v.at[b], sem_g).wait()
            pltpu.make_async_copy(R_hbm.at[r_v.at[j]], rr_v.at[b], sem_g).wait()

        def drain_store(j, b):
            if write_e:
                pltpu.make_async_copy(
                    e_v.at[b], e_out.at[t, pl.ds(j * CH, CH)], sem_sl).wait()
            pltpu.make_async_copy(
                e_v.at[b], agg_sh.at[r_v.at[j]], sem_si).wait()

        def compute(j, b):
            @pl.loop(0, CH, unroll=8)
            def _row(i):
                for h in range(2):
                    sl = pl.ds(h * 16, 16)
                    e_v[b, i, sl] = jnp.maximum(
                        b_v[b, i, sl] + sr_v[b, i, sl] + rr_v[b, i, sl], 0.0
                    )

        def issue_store(j, b):
            if write_e:
                pltpu.async_copy(
                    e_v.at[b], e_out.at[t, pl.ds(j * CH, CH)], sem_sl)
            pltpu.async_copy(e_v.at[b], agg_sh.at[r_v.at[j]], sem_si, add=True)

        issue_in(0, 0)

        @pl.loop(0, NCH // 2)
        def _pair(jj):
            for par in range(2):
                j = 2 * jj + par
                nb = 1 - par
                issue_in(j + 1, nb)
                wait_in(j, par)

                @pl.when(j >= 2)
                def _():
                    drain_store(j, par)

                compute(j, par)
                issue_store(j, par)

        jl = NCH - 1
        wait_in(jl, 0)
        drain_store(jl - 2, 0)
        compute(jl, 0)
        issue_store(jl, 0)
        drain_store(jl - 1, 1)
        drain_store(jl, 0)

        plsc.subcore_barrier()
        pltpu.sync_copy(
            agg_sh.at[pl.ds(sid * NROWS, NROWS)],
            agg_out.at[c, pl.ds(sid * NROWS, NROWS)],
        )

    return pl.kernel(
        body,
        out_type=tuple(out_type),
        mesh=mesh,
        scratch_types=scratch,
        compiler_params=pltpu.CompilerParams(use_tc_tiling_on_sc=False),
    )


_sc_block_we = _make_sc_block(True)
_sc_block_agg = _make_sc_block(False)


_GRID = 25


def _mm(a, w):
    return jnp.dot(a, w, preferred_element_type=jnp.float32,
                   precision=jax.lax.Precision.HIGHEST)


def _stage(groups):
    arrays = []
    specs = []
    out_specs, out_shape = [], []
    plans = []

    def add(arr, spec):
        arrays.append(arr)
        specs.append(spec)
        return len(arrays) - 1

    for g in groups:
        if "t4" in g:
            eaT, W, bias = g["t4"]
            K = eaT.shape[0]
            colsb = E4 // _GRID
            panel_ids = [
                add(eaT, pl.BlockSpec((K, colsb),
                                      lambda i, c=c: (0, c * _GRID + i)))
                for c in range(4)
            ]
            w_id = add(W, pl.BlockSpec(W.shape, lambda i: (0, 0)))
            b = bias.reshape(1, -1)
            b_id = add(b, pl.BlockSpec(b.shape, lambda i: (0, 0)))
            out_specs.append(pl.BlockSpec((colsb, 128), lambda i: (i, 0)))
            out_shape.append(jax.ShapeDtypeStruct((E4, 128), jnp.float32))
            plans.append(("t4", panel_ids, w_id, b_id))
        else:
            rows = g["ins"][0].shape[0] if g["ins"] else N
            br = (E4 if rows == E4 else N) // _GRID
            in_ids = [
                add(a, pl.BlockSpec((br, a.shape[1]), lambda i: (i, 0)))
                for a in g["ins"]
            ]
            outs = []
            for terms, bias, relu in g["outs"]:
                t_ids = [
                    (kind, idx,
                     add(W, pl.BlockSpec(W.shape, lambda i: (0, 0))))
                    for kind, idx, W in terms
                ]
                b = bias.reshape(1, -1)
                b_id = add(b, pl.BlockSpec(b.shape, lambda i: (0, 0)))
                out_specs.append(pl.BlockSpec((br, b.shape[1]),
                                              lambda i: (i, 0)))
                out_shape.append(
                    jax.ShapeDtypeStruct((br * _GRID, b.shape[1]),
                                         jnp.float32))
                outs.append((t_ids, b_id, relu))
            plans.append(("gen", in_ids, outs))

    n_in = len(arrays)

    def body(*refs):
        in_refs = refs[:n_in]
        o_refs = refs[n_in:]
        oi = 0
        for plan in plans:
            if plan[0] == "t4":
                _, panel_ids, w_id, b_id = plan
                w = in_refs[w_id][...]
                vals = [
                    lax.dot_general(
                        in_refs[p][...], w, (((0,), (0,)), ((), ())),
                        preferred_element_type=jnp.float32,
                        precision=jax.lax.Precision.HIGHEST)
                    for p in panel_ids
                ]
                o_refs[oi][...] = jnp.concatenate(vals, axis=1) \
                    + in_refs[b_id][...]
                oi += 1
            else:
                _, in_ids, outs = plan
                outvals = []
                for t_ids, b_id, relu in outs:
                    acc = in_refs[b_id][...]
                    for kind, idx, w_id in t_ids:
                        op = (in_refs[in_ids[idx]][...] if kind == "in"
                              else outvals[idx])
                        acc = acc + _mm(op, in_refs[w_id][...])
                    val = jnp.maximum(acc, 0.0) if relu else acc
                    outvals.append(val)
                    o_refs[oi][...] = val
                    oi += 1

    return pl.pallas_call(
        body,
        grid=(_GRID,),
        in_specs=specs,
        out_specs=out_specs,
        out_shape=out_shape,
    )(*arrays)


def _blockdiag(W, k):
    din, dout = W.shape
    out = jnp.zeros((k * din, k * dout), jnp.float32)
    for i in range(k):
        out = out.at[i * din:(i + 1) * din, i * dout:(i + 1) * dout].set(W)
    return out


def kernel(x, edge_attr, edge_index,
           enc_We, enc_be, enc_Wn, enc_bn,
           proc_We, proc_be, proc_Wn, proc_bn,
           dec_We, dec_be, dec_Wn, dec_bn):
    q = jnp.arange(E, dtype=jnp.int32)
    perm = (q % 4) * E4 + q // 4
    s3 = edge_index[0][perm].reshape(NW, NCH, CH)
    r3 = edge_index[1][perm].reshape(NW, NCH, CH)
    zeros = jnp.zeros((NP, L), jnp.float32)

    We_e = enc_We[:DE]
    We_s = enc_We[DE:DE + DF]
    We_r = enc_We[DE + DF:]
    W_ce = proc_We[0 * L:1 * L]
    W_ee = proc_We[1 * L:2 * L]
    W_scx = proc_We[2 * L:3 * L]
    W_sex = proc_We[3 * L:4 * L]
    W_rcx = proc_We[4 * L:5 * L]
    W_rex = proc_We[5 * L:6 * L]
    Wn_cx = proc_Wn[0 * L:1 * L]
    Wn_ex = proc_Wn[1 * L:2 * L]
    Wn_agg = proc_Wn[2 * L:3 * L]

    bd = functools.partial(_blockdiag, k=4)
    be4 = lambda b: jnp.tile(b, 4)

    z32 = jnp.zeros((L,), jnp.float32)

    base1, S1, R1 = _stage([
        {"t4": (edge_attr.T, We_e, be4(enc_be))},
        {"ins": [x],
         "outs": [
             ([("in", 0, We_s)], z32, False),
             ([("in", 0, We_r)], z32, False),
         ]},
    ])

    he3, agg1p = _sc_block_we(s3, r3, base1.reshape(NW, EW, L), S1, R1, zeros)
    he4 = he3.reshape(E4, 128)

    base2, hx, S2, R2 = _stage([
        {"ins": [he4],
         "outs": [([("in", 0, bd(W_ce + W_ee))], be4(proc_be), False)]},
        {"ins": [x, agg1p[0], agg1p[1]],
         "outs": [
             ([("in", 0, enc_Wn[:DF]), ("in", 1, enc_Wn[DF:]),
               ("in", 2, enc_Wn[DF:])], enc_bn, True),
             ([("out", 0, W_scx + W_sex)], z32, False),
             ([("out", 0, W_rcx + W_rex)], z32, False),
         ]},
    ])

    ce13, agg2p = _sc_block_we(s3, r3, base2.reshape(NW, EW, L), S2, R2, zeros)
    ce14 = ce13.reshape(E4, 128)

    base3, cx1, S3, R3 = _stage([
        {"ins": [ce14, he4],
         "outs": [([("in", 0, bd(W_ce)), ("in", 1, bd(W_ee))],
                   be4(proc_be), False)]},
        {"ins": [hx, agg2p[0], agg2p[1]],
         "outs": [
             ([("in", 0, Wn_cx + Wn_ex), ("in", 1, Wn_agg),
               ("in", 2, Wn_agg)], proc_bn, True),
             ([("out", 0, W_scx), ("in", 0, W_sex)], z32, False),
             ([("out", 0, W_rcx), ("in", 0, W_rex)], z32, False),
         ]},
    ])

    ce23, agg3p = _sc_block_we(s3, r3, base3.reshape(NW, EW, L), S3, R3, zeros)
    ce24 = ce23.reshape(E4, 128)

    base4, cx2, S4, R4 = _stage([
        {"ins": [ce24],
         "outs": [([("in", 0, bd(dec_We[:L]))], be4(dec_be), False)]},
        {"ins": [cx1, hx, agg3p[0], agg3p[1]],
         "outs": [
             ([("in", 0, Wn_cx), ("in", 1, Wn_ex), ("in", 2, Wn_agg),
               ("in", 3, Wn_agg)], proc_bn, True),
             ([("out", 0, dec_We[L:2 * L])], z32, False),
             ([("out", 0, dec_We[2 * L:])], z32, False),
         ]},
    ])

    (agg4p,) = _sc_block_agg(s3, r3, base4.reshape(NW, EW, L), S4, R4, zeros)

    (out_x,) = _stage([
        {"ins": [cx2, agg4p[0], agg4p[1]],
         "outs": [
             ([("in", 0, dec_Wn[:L]), ("in", 1, dec_Wn[L:]),
               ("in", 2, dec_Wn[L:])], dec_bn, True),
         ]},
    ])
    return out_x


# speedup vs baseline: 1.1490x; 1.1490x over previous
import functools

import jax
import jax.numpy as jnp
from jax import lax
from jax.experimental import pallas as pl
from jax.experimental.pallas import tpu as pltpu
from jax.experimental.pallas import tpu_sc as plsc

N = 10000
E = 320000
DF = 128
DE = 16
L = 32

NC = 2
NS = 16
NW = NC * NS
EW = E // NW
CH = 80
NCH = EW // CH
NP = 10240
NROWS = NP // NS

E4 = E // 4


def _make_sc_block(write_e: bool):
    mesh = plsc.VectorSubcoreMesh(
        core_axis_name="c", subcore_axis_name="s", num_cores=NC, num_subcores=NS
    )
    out_type = []
    if write_e:
        out_type.append(jax.ShapeDtypeStruct((NW, EW, L), jnp.float32))
    out_type.append(jax.ShapeDtypeStruct((NC, NP, L), jnp.float32))

    scratch = [
        pltpu.VMEM((NCH, CH), jnp.int32),
        pltpu.VMEM((NCH, CH), jnp.int32),
        pltpu.VMEM((2, CH, L), jnp.float32),
        pltpu.VMEM((2, CH, L), jnp.float32),
        pltpu.VMEM((2, CH, L), jnp.float32),
        pltpu.VMEM((2, CH, L), jnp.float32),
        pltpu.SemaphoreType.DMA,
        pltpu.SemaphoreType.DMA,
        pltpu.SemaphoreType.DMA,
        pltpu.SemaphoreType.DMA,
        pltpu.VMEM_SHARED((NP, L), jnp.float32),
    ]

    def body(s_hbm, r_hbm, base_hbm, S_hbm, R_hbm, z_hbm, *rest):
        if write_e:
            e_out, agg_out = rest[0], rest[1]
            scr = rest[2:]
        else:
            agg_out = rest[0]
            scr = rest[1:]
        s_v, r_v, b_v, sr_v, rr_v, e_v, sem_b, sem_g, sem_sl, sem_si, agg_sh = scr

        c = lax.axis_index("c")
        sid = lax.axis_index("s")
        t = c * NS + sid

        pltpu.sync_copy(
            z_hbm.at[pl.ds(sid * NROWS, NROWS)],
            agg_sh.at[pl.ds(sid * NROWS, NROWS)],
        )
        pltpu.sync_copy(s_hbm.at[t], s_v)
        pltpu.sync_copy(r_hbm.at[t], r_v)
        plsc.subcore_barrier()

        def issue_in(j, b):
            pltpu.async_copy(base_hbm.at[t, pl.ds(j * CH, CH)], b_v.at[b], sem_b)
            pltpu.async_copy(S_hbm.at[s_v.at[j]], sr_v.at[b], sem_g)
            pltpu.async_copy(R_hbm.at[r_v.at[j]], rr_v.at[b], sem_g)

        def wait_in(j, b):
            pltpu.make_async_copy(
                base_hbm.at[t, pl.ds(j * CH, CH)], b_v.at[b], sem_b).wait()
            pltpu.make_async_copy(S_hbm.at[s_v.at[j]], sr_v.at[b], sem_g).wait()
            pltpu.make_async_copy(R_hbm.at[r_v.at[j]], rr_v.at[b], sem_g).wait()

        def drain_store(j, b):
            if write_e:
                pltpu.make_async_copy(
                    e_v.at[b], e_out.at[t, pl.ds(j * CH, CH)], sem_sl).wait()
            pltpu.make_async_copy(
                e_v.at[b], agg_sh.at[r_v.at[j]], sem_si).wait()

        def compute(j, b):
            @pl.loop(0, CH, unroll=8)
            def _row(i):
                for h in range(2):
                    sl = pl.ds(h * 16, 16)
                    e_v[b, i, sl] = jnp.maximum(
                        b_v[b, i, sl] + sr_v[b, i, sl] + rr_v[b, i, sl], 0.0
                    )

        def issue_store(j, b):
            if write_e:
                pltpu.async_copy(
                    e_v.at[b], e_out.at[t, pl.ds(j * CH, CH)], sem_sl)
            pltpu.async_copy(e_v.at[b], agg_sh.at[r_v.at[j]], sem_si, add=True)

        issue_in(0, 0)

        @pl.loop(0, NCH // 2)
        def _pair(jj):
            for par in range(2):
                j = 2 * jj + par
                nb = 1 - par
                issue_in(j + 1, nb)
                wait_in(j, par)

                @pl.when(j >= 2)
                def _():
                    drain_store(j, par)

                compute(j, par)
                issue_store(j, par)

        jl = NCH - 1
        wait_in(jl, 0)
        drain_store(jl - 2, 0)
        compute(jl, 0)
        issue_store(jl, 0)
        drain_store(jl - 1, 1)
        drain_store(jl, 0)

        plsc.subcore_barrier()
        pltpu.sync_copy(
            agg_sh.at[pl.ds(sid * NROWS, NROWS)],
            agg_out.at[c, pl.ds(sid * NROWS, NROWS)],
        )

    return pl.kernel(
        body,
        out_type=tuple(out_type),
        mesh=mesh,
        scratch_types=scratch,
        compiler_params=pltpu.CompilerParams(use_tc_tiling_on_sc=False),
    )


_sc_block_we = _make_sc_block(True)
_sc_block_agg = _make_sc_block(False)


_GRID = 25


def _mm(a, w):
    return jnp.dot(a, w, preferred_element_type=jnp.float32)


def _stage(groups):
    arrays = []
    specs = []
    out_specs, out_shape = [], []
    plans = []

    def add(arr, spec):
        arrays.append(arr)
        specs.append(spec)
        return len(arrays) - 1

    for g in groups:
        if "t4" in g:
            eaT, W, bias = g["t4"]
            K = eaT.shape[0]
            colsb = E4 // _GRID
            panel_ids = [
                add(eaT, pl.BlockSpec((K, colsb),
                                      lambda i, c=c: (0, c * _GRID + i)))
                for c in range(4)
            ]
            w_id = add(W, pl.BlockSpec(W.shape, lambda i: (0, 0)))
            b = bias.reshape(1, -1)
            b_id = add(b, pl.BlockSpec(b.shape, lambda i: (0, 0)))
            out_specs.append(pl.BlockSpec((colsb, 128), lambda i: (i, 0)))
            out_shape.append(jax.ShapeDtypeStruct((E4, 128), jnp.float32))
            plans.append(("t4", panel_ids, w_id, b_id))
        else:
            rows = g["ins"][0].shape[0] if g["ins"] else N
            br = (E4 if rows == E4 else N) // _GRID
            in_ids = [
                add(a, pl.BlockSpec((br, a.shape[1]), lambda i: (i, 0)))
                for a in g["ins"]
            ]
            outs = []
            for terms, bias, relu in g["outs"]:
                t_ids = [
                    (kind, idx,
                     add(W, pl.BlockSpec(W.shape, lambda i: (0, 0))))
                    for kind, idx, W in terms
                ]
                b = bias.reshape(1, -1)
                b_id = add(b, pl.BlockSpec(b.shape, lambda i: (0, 0)))
                out_specs.append(pl.BlockSpec((br, b.shape[1]),
                                              lambda i: (i, 0)))
                out_shape.append(
                    jax.ShapeDtypeStruct((br * _GRID, b.shape[1]),
                                         jnp.float32))
                outs.append((t_ids, b_id, relu))
            plans.append(("gen", in_ids, outs))

    n_in = len(arrays)

    def body(*refs):
        in_refs = refs[:n_in]
        o_refs = refs[n_in:]
        oi = 0
        for plan in plans:
            if plan[0] == "t4":
                _, panel_ids, w_id, b_id = plan
                w = in_refs[w_id][...]
                vals = [
                    lax.dot_general(
                        in_refs[p][...], w, (((0,), (0,)), ((), ())),
                        preferred_element_type=jnp.float32)
                    for p in panel_ids
                ]
                o_refs[oi][...] = jnp.concatenate(vals, axis=1) \
                    + in_refs[b_id][...]
                oi += 1
            else:
                _, in_ids, outs = plan
                outvals = []
                for t_ids, b_id, relu in outs:
                    acc = in_refs[b_id][...]
                    for kind, idx, w_id in t_ids:
                        op = (in_refs[in_ids[idx]][...] if kind == "in"
                              else outvals[idx])
                        acc = acc + _mm(op, in_refs[w_id][...])
                    val = jnp.maximum(acc, 0.0) if relu else acc
                    outvals.append(val)
                    o_refs[oi][...] = val
                    oi += 1

    return pl.pallas_call(
        body,
        grid=(_GRID,),
        in_specs=specs,
        out_specs=out_specs,
        out_shape=out_shape,
        compiler_params=pltpu.CompilerParams(
            fuse_transposed_lhs_in_matmul=True),
    )(*arrays)


def _blockdiag(W, k):
    din, dout = W.shape
    out = jnp.zeros((k * din, k * dout), jnp.float32)
    for i in range(k):
        out = out.at[i * din:(i + 1) * din, i * dout:(i + 1) * dout].set(W)
    return out


def kernel(x, edge_attr, edge_index,
           enc_We, enc_be, enc_Wn, enc_bn,
           proc_We, proc_be, proc_Wn, proc_bn,
           dec_We, dec_be, dec_Wn, dec_bn):
    s3 = edge_index[0].reshape(4, E4).T.reshape(NW, NCH, CH)
    r3 = edge_index[1].reshape(4, E4).T.reshape(NW, NCH, CH)
    zeros = jnp.zeros((NP, L), jnp.float32)

    We_e = enc_We[:DE]
    We_s = enc_We[DE:DE + DF]
    We_r = enc_We[DE + DF:]
    W_ce = proc_We[0 * L:1 * L]
    W_ee = proc_We[1 * L:2 * L]
    W_scx = proc_We[2 * L:3 * L]
    W_sex = proc_We[3 * L:4 * L]
    W_rcx = proc_We[4 * L:5 * L]
    W_rex = proc_We[5 * L:6 * L]
    Wn_cx = proc_Wn[0 * L:1 * L]
    Wn_ex = proc_Wn[1 * L:2 * L]
    Wn_agg = proc_Wn[2 * L:3 * L]

    bd = functools.partial(_blockdiag, k=4)
    be4 = lambda b: jnp.tile(b, 4)

    z32 = jnp.zeros((L,), jnp.float32)

    base1, S1, R1 = _stage([
        {"t4": (edge_attr.T, We_e, be4(enc_be))},
        {"ins": [x],
         "outs": [
             ([("in", 0, We_s)], z32, False),
             ([("in", 0, We_r)], z32, False),
         ]},
    ])

    he3, agg1p = _sc_block_we(s3, r3, base1.reshape(NW, EW, L), S1, R1, zeros)
    he4 = he3.reshape(E4, 128)

    base2, hx, S2, R2 = _stage([
        {"ins": [he4],
         "outs": [([("in", 0, bd(W_ce + W_ee))], be4(proc_be), False)]},
        {"ins": [x, agg1p[0], agg1p[1]],
         "outs": [
             ([("in", 0, enc_Wn[:DF]), ("in", 1, enc_Wn[DF:]),
               ("in", 2, enc_Wn[DF:])], enc_bn, True),
             ([("out", 0, W_scx + W_sex)], z32, False),
             ([("out", 0, W_rcx + W_rex)], z32, False),
         ]},
    ])

    ce13, agg2p = _sc_block_we(s3, r3, base2.reshape(NW, EW, L), S2, R2, zeros)
    ce14 = ce13.reshape(E4, 128)

    base3, cx1, S3, R3 = _stage([
        {"ins": [ce14, he4],
         "outs": [([("in", 0, bd(W_ce)), ("in", 1, bd(W_ee))],
                   be4(proc_be), False)]},
        {"ins": [hx, agg2p[0], agg2p[1]],
         "outs": [
             ([("in", 0, Wn_cx + Wn_ex), ("in", 1, Wn_agg),
               ("in", 2, Wn_agg)], proc_bn, True),
             ([("out", 0, W_scx), ("in", 0, W_sex)], z32, False),
             ([("out", 0, W_rcx), ("in", 0, W_rex)], z32, False),
         ]},
    ])

    ce23, agg3p = _sc_block_we(s3, r3, base3.reshape(NW, EW, L), S3, R3, zeros)
    ce24 = ce23.reshape(E4, 128)

    base4, cx2, S4, R4 = _stage([
        {"ins": [ce24],
         "outs": [([("in", 0, bd(dec_We[:L]))], be4(dec_be), False)]},
        {"ins": [cx1, hx, agg3p[0], agg3p[1]],
         "outs": [
             ([("in", 0, Wn_cx), ("in", 1, Wn_ex), ("in", 2, Wn_agg),
               ("in", 3, Wn_agg)], proc_bn, True),
             ([("out", 0, dec_We[L:2 * L])], z32, False),
             ([("out", 0, dec_We[2 * L:])], z32, False),
         ]},
    ])

    (agg4p,) = _sc_block_agg(s3, r3, base4.reshape(NW, EW, L), S4, R4, zeros)

    (out_x,) = _stage([
        {"ins": [cx2, agg4p[0], agg4p[1]],
         "outs": [
             ([("in", 0, dec_Wn[:L]), ("in", 1, dec_Wn[L:]),
               ("in", 2, dec_Wn[L:])], dec_bn, True),
         ]},
    ])
    return out_x


# speedup vs baseline: 1.1786x; 1.0258x over previous
import functools

import jax
import jax.numpy as jnp
from jax import lax
from jax.experimental import pallas as pl
from jax.experimental.pallas import tpu as pltpu
from jax.experimental.pallas import tpu_sc as plsc

N = 10000
E = 320000
DF = 128
DE = 16
L = 32

NC = 2
NS = 16
NW = NC * NS
EW = E // NW
CH = 80
NCH = EW // CH
NP = 10240
NROWS = NP // NS

E4 = E // 4


def _make_sc_block(write_e: bool):
    mesh = plsc.VectorSubcoreMesh(
        core_axis_name="c", subcore_axis_name="s", num_cores=NC, num_subcores=NS
    )
    out_type = []
    if write_e:
        out_type.append(jax.ShapeDtypeStruct((NW, EW, L), jnp.float32))
    out_type.append(jax.ShapeDtypeStruct((NC, NP, L), jnp.float32))

    scratch = [
        pltpu.VMEM((NCH, CH), jnp.int32),
        pltpu.VMEM((NCH, CH), jnp.int32),
        pltpu.VMEM((2, CH, L), jnp.float32),
        pltpu.VMEM((2, CH, L), jnp.float32),
        pltpu.VMEM((2, CH, L), jnp.float32),
        pltpu.VMEM((2, CH, L), jnp.float32),
        pltpu.SemaphoreType.DMA,
        pltpu.SemaphoreType.DMA,
        pltpu.SemaphoreType.DMA,
        pltpu.SemaphoreType.DMA,
        pltpu.VMEM_SHARED((NP, L), jnp.float32),
    ]

    def body(s_hbm, r_hbm, base_hbm, S_hbm, R_hbm, z_hbm, *rest):
        if write_e:
            e_out, agg_out = rest[0], rest[1]
            scr = rest[2:]
        else:
            agg_out = rest[0]
            scr = rest[1:]
        s_v, r_v, b_v, sr_v, rr_v, e_v, sem_b, sem_g, sem_sl, sem_si, agg_sh = scr

        c = lax.axis_index("c")
        sid = lax.axis_index("s")
        t = c * NS + sid

        pltpu.sync_copy(
            z_hbm.at[pl.ds(sid * NROWS, NROWS)],
            agg_sh.at[pl.ds(sid * NROWS, NROWS)],
        )
        pltpu.sync_copy(s_hbm.at[t], s_v)
        pltpu.sync_copy(r_hbm.at[t], r_v)
        plsc.subcore_barrier()

        def issue_in(j, b):
            pltpu.async_copy(base_hbm.at[t, pl.ds(j * CH, CH)], b_v.at[b], sem_b)
            pltpu.async_copy(S_hbm.at[s_v.at[j]], sr_v.at[b], sem_g)
            pltpu.async_copy(R_hbm.at[r_v.at[j]], rr_v.at[b], sem_g)

        def wait_in(j, b):
            pltpu.make_async_copy(
                base_hbm.at[t, pl.ds(j * CH, CH)], b_v.at[b], sem_b).wait()
            pltpu.make_async_copy(S_hbm.at[s_v.at[j]], sr_v.at[b], sem_g).wait()
            pltpu.make_async_copy(R_hbm.at[r_v.at[j]], rr_v.at[b], sem_g).wait()

        def drain_store(j, b):
            if write_e:
                pltpu.make_async_copy(
                    e_v.at[b], e_out.at[t, pl.ds(j * CH, CH)], sem_sl).wait()
            pltpu.make_async_copy(
                e_v.at[b], agg_sh.at[r_v.at[j]], sem_si).wait()

        def compute(j, b):
            @pl.loop(0, CH, unroll=8)
            def _row(i):
                for h in range(2):
                    sl = pl.ds(h * 16, 16)
                    e_v[b, i, sl] = jnp.maximum(
                        b_v[b, i, sl] + sr_v[b, i, sl] + rr_v[b, i, sl], 0.0
                    )

        def issue_store(j, b):
            if write_e:
                pltpu.async_copy(
                    e_v.at[b], e_out.at[t, pl.ds(j * CH, CH)], sem_sl)
            pltpu.async_copy(e_v.at[b], agg_sh.at[r_v.at[j]], sem_si, add=True)

        issue_in(0, 0)

        @pl.loop(0, NCH // 2)
        def _pair(jj):
            for par in range(2):
                j = 2 * jj + par
                nb = 1 - par
                issue_in(j + 1, nb)
                wait_in(j, par)

                @pl.when(j >= 2)
                def _():
                    drain_store(j, par)

                compute(j, par)
                issue_store(j, par)

        jl = NCH - 1
        wait_in(jl, 0)
        drain_store(jl - 2, 0)
        compute(jl, 0)
        issue_store(jl, 0)
        drain_store(jl - 1, 1)
        drain_store(jl, 0)

        plsc.subcore_barrier()
        pltpu.sync_copy(
            agg_sh.at[pl.ds(sid * NROWS, NROWS)],
            agg_out.at[c, pl.ds(sid * NROWS, NROWS)],
        )

    return pl.kernel(
        body,
        out_type=tuple(out_type),
        mesh=mesh,
        scratch_types=scratch,
        compiler_params=pltpu.CompilerParams(use_tc_tiling_on_sc=False),
    )


_sc_block_we = _make_sc_block(True)
_sc_block_agg = _make_sc_block(False)


NPK = NP // 4


def _mm(a, w):
    return jnp.dot(a, w, preferred_element_type=jnp.float32)


def _stage(groups, grid):
    arrays = []
    specs = []
    out_specs, out_shape = [], []
    plans = []

    def add(arr, spec):
        arrays.append(arr)
        specs.append(spec)
        return len(arrays) - 1

    for g in groups:
        if "t4" in g:
            eaT, W, bias = g["t4"]
            K = eaT.shape[0]
            colsb = E4 // grid
            panel_ids = [
                add(eaT, pl.BlockSpec((K, colsb),
                                      lambda i, c=c: (0, c * grid + i)))
                for c in range(4)
            ]
            w_id = add(W, pl.BlockSpec(W.shape, lambda i: (0, 0)))
            b = bias.reshape(1, -1)
            b_id = add(b, pl.BlockSpec(b.shape, lambda i: (0, 0)))
            out_specs.append(pl.BlockSpec((colsb, 128), lambda i: (i, 0)))
            out_shape.append(jax.ShapeDtypeStruct((E4, 128), jnp.float32))
            plans.append(("t4", panel_ids, w_id, b_id))
        else:
            rows = g["ins"][0].shape[0] if g["ins"] else NPK
            br = (E4 if rows == E4 else NPK) // grid
            in_ids = [
                add(a, pl.BlockSpec((br, a.shape[1]), lambda i: (i, 0)))
                for a in g["ins"]
            ]
            outs = []
            for terms, bias, relu in g["outs"]:
                t_ids = [
                    (kind, idx,
                     add(W, pl.BlockSpec(W.shape, lambda i: (0, 0))))
                    for kind, idx, W in terms
                ]
                b = bias.reshape(1, -1)
                b_id = add(b, pl.BlockSpec(b.shape, lambda i: (0, 0)))
                out_specs.append(pl.BlockSpec((br, b.shape[1]),
                                              lambda i: (i, 0)))
                out_shape.append(
                    jax.ShapeDtypeStruct((br * grid, b.shape[1]),
                                         jnp.float32))
                outs.append((t_ids, b_id, relu))
            plans.append(("gen", in_ids, outs))

    n_in = len(arrays)

    def body(*refs):
        in_refs = refs[:n_in]
        o_refs = refs[n_in:]
        oi = 0
        for plan in plans:
            if plan[0] == "t4":
                _, panel_ids, w_id, b_id = plan
                w = in_refs[w_id][...]
                vals = [
                    lax.dot_general(
                        in_refs[p][...], w, (((0,), (0,)), ((), ())),
                        preferred_element_type=jnp.float32)
                    for p in panel_ids
                ]
                o_refs[oi][...] = jnp.concatenate(vals, axis=1) \
                    + in_refs[b_id][...]
                oi += 1
            else:
                _, in_ids, outs = plan
                outvals = []
                for t_ids, b_id, relu in outs:
                    acc = in_refs[b_id][...]
                    for kind, idx, w_id in t_ids:
                        op = (in_refs[in_ids[idx]][...] if kind == "in"
                              else outvals[idx])
                        acc = acc + _mm(op, in_refs[w_id][...])
                    val = jnp.maximum(acc, 0.0) if relu else acc
                    outvals.append(val)
                    o_refs[oi][...] = val
                    oi += 1

    return pl.pallas_call(
        body,
        grid=(grid,),
        in_specs=specs,
        out_specs=out_specs,
        out_shape=out_shape,
        compiler_params=pltpu.CompilerParams(
            fuse_transposed_lhs_in_matmul=True),
    )(*arrays)


def _blockdiag(W, k):
    din, dout = W.shape
    out = jnp.zeros((k * din, k * dout), jnp.float32)
    for i in range(k):
        out = out.at[i * din:(i + 1) * din, i * dout:(i + 1) * dout].set(W)
    return out


def kernel(x, edge_attr, edge_index,
           enc_We, enc_be, enc_Wn, enc_bn,
           proc_We, proc_be, proc_Wn, proc_bn,
           dec_We, dec_be, dec_Wn, dec_bn):
    s3 = edge_index[0].reshape(4, E4).T.reshape(NW, NCH, CH)
    r3 = edge_index[1].reshape(4, E4).T.reshape(NW, NCH, CH)
    zeros = jnp.zeros((NP, L), jnp.float32)

    We_e = enc_We[:DE]
    We_s = enc_We[DE:DE + DF]
    We_r = enc_We[DE + DF:]
    W_ce = proc_We[0 * L:1 * L]
    W_ee = proc_We[1 * L:2 * L]
    W_scx = proc_We[2 * L:3 * L]
    W_sex = proc_We[3 * L:4 * L]
    W_rcx = proc_We[4 * L:5 * L]
    W_rex = proc_We[5 * L:6 * L]
    Wn_cx = proc_Wn[0 * L:1 * L]
    Wn_ex = proc_Wn[1 * L:2 * L]
    Wn_agg = proc_Wn[2 * L:3 * L]

    bd = functools.partial(_blockdiag, k=4)
    be4 = lambda b: jnp.tile(b, 4)

    z128 = jnp.zeros((128,), jnp.float32)

    xp = jnp.concatenate([x, jnp.zeros((NP - N, DF), jnp.float32)])
    x4 = xp.reshape(NPK, 4 * DF)
    pk = lambda a: a.reshape(NPK, 128)
    tb = lambda a: a.reshape(NP, L)

    base1, S1, R1 = _stage([
        {"t4": (edge_attr.T, We_e, be4(enc_be))},
        {"ins": [x4],
         "outs": [
             ([("in", 0, bd(We_s))], z128, False),
             ([("in", 0, bd(We_r))], z128, False),
         ]},
    ], 5)

    he3, agg1p = _sc_block_we(s3, r3, base1.reshape(NW, EW, L),
                              tb(S1), tb(R1), zeros)
    he4 = he3.reshape(E4, 128)

    base2, hx, S2, R2 = _stage([
        {"ins": [he4],
         "outs": [([("in", 0, bd(W_ce + W_ee))], be4(proc_be), False)]},
        {"ins": [x4, pk(agg1p[0]), pk(agg1p[1])],
         "outs": [
             ([("in", 0, bd(enc_Wn[:DF])), ("in", 1, bd(enc_Wn[DF:])),
               ("in", 2, bd(enc_Wn[DF:]))], be4(enc_bn), True),
             ([("out", 0, bd(W_scx + W_sex))], z128, False),
             ([("out", 0, bd(W_rcx + W_rex))], z128, False),
         ]},
    ], 20)

    ce13, agg2p = _sc_block_we(s3, r3, base2.reshape(NW, EW, L),
                               tb(S2), tb(R2), zeros)
    ce14 = ce13.reshape(E4, 128)

    base3, cx1, S3, R3 = _stage([
        {"ins": [ce14, he4],
         "outs": [([("in", 0, bd(W_ce)), ("in", 1, bd(W_ee))],
                   be4(proc_be), False)]},
        {"ins": [hx, pk(agg2p[0]), pk(agg2p[1])],
         "outs": [
             ([("in", 0, bd(Wn_cx + Wn_ex)), ("in", 1, bd(Wn_agg)),
               ("in", 2, bd(Wn_agg))], be4(proc_bn), True),
             ([("out", 0, bd(W_scx)), ("in", 0, bd(W_sex))], z128, False),
             ([("out", 0, bd(W_rcx)), ("in", 0, bd(W_rex))], z128, False),
         ]},
    ], 20)

    ce23, agg3p = _sc_block_we(s3, r3, base3.reshape(NW, EW, L),
                               tb(S3), tb(R3), zeros)
    ce24 = ce23.reshape(E4, 128)

    base4, cx2, S4, R4 = _stage([
        {"ins": [ce24],
         "outs": [([("in", 0, bd(dec_We[:L]))], be4(dec_be), False)]},
        {"ins": [cx1, hx, pk(agg3p[0]), pk(agg3p[1])],
         "outs": [
             ([("in", 0, bd(Wn_cx)), ("in", 1, bd(Wn_ex)),
               ("in", 2, bd(Wn_agg)), ("in", 3, bd(Wn_agg))],
              be4(proc_bn), True),
             ([("out", 0, bd(dec_We[L:2 * L]))], z128, False),
             ([("out", 0, bd(dec_We[2 * L:]))], z128, False),
         ]},
    ], 20)

    (agg4p,) = _sc_block_agg(s3, r3, base4.reshape(NW, EW, L),
                             tb(S4), tb(R4), zeros)

    (outp,) = _stage([
        {"ins": [cx2, pk(agg4p[0]), pk(agg4p[1])],
         "outs": [
             ([("in", 0, bd(dec_Wn[:L])), ("in", 1, bd(dec_Wn[L:])),
               ("in", 2, bd(dec_Wn[L:]))], be4(dec_bn), True),
         ]},
    ], 10)
    return outp.reshape(NP, L)[:N]


# speedup vs baseline: 1.1847x; 1.0051x over previous
import functools

import jax
import jax.numpy as jnp
from jax import lax
from jax.experimental import pallas as pl
from jax.experimental.pallas import tpu as pltpu
from jax.experimental.pallas import tpu_sc as plsc

N = 10000
E = 320000
DF = 128
DE = 16
L = 32

NC = 2
NS = 16
NW = NC * NS
EW = E // NW
CH = 80
NCH = EW // CH
NP = 10240
NROWS = NP // NS

E4 = E // 4


def _make_sc_block(write_e: bool):
    mesh = plsc.VectorSubcoreMesh(
        core_axis_name="c", subcore_axis_name="s", num_cores=NC, num_subcores=NS
    )
    out_type = []
    if write_e:
        out_type.append(jax.ShapeDtypeStruct((NW, EW, L), jnp.float32))
    out_type.append(jax.ShapeDtypeStruct((NP, L), jnp.float32))
    out_type.append(jax.ShapeDtypeStruct((NP, L), jnp.float32))

    scratch = [
        pltpu.VMEM((NCH, CH), jnp.int32),
        pltpu.VMEM((NCH, CH), jnp.int32),
        pltpu.VMEM((2, CH, L), jnp.float32),
        pltpu.VMEM((2, CH, L), jnp.float32),
        pltpu.VMEM((2, CH, L), jnp.float32),
        pltpu.VMEM((2, CH, L), jnp.float32),
        pltpu.SemaphoreType.DMA,
        pltpu.SemaphoreType.DMA,
        pltpu.SemaphoreType.DMA,
        pltpu.SemaphoreType.DMA,
        pltpu.VMEM_SHARED((NP, L), jnp.float32),
    ]

    def body(s_hbm, r_hbm, base_hbm, S_hbm, R_hbm, z_hbm, *rest):
        if write_e:
            e_out, agg_out0, agg_out1 = rest[0], rest[1], rest[2]
            scr = rest[3:]
        else:
            agg_out0, agg_out1 = rest[0], rest[1]
            scr = rest[2:]
        s_v, r_v, b_v, sr_v, rr_v, e_v, sem_b, sem_g, sem_sl, sem_si, agg_sh = scr

        c = lax.axis_index("c")
        sid = lax.axis_index("s")
        t = c * NS + sid

        pltpu.sync_copy(
            z_hbm.at[pl.ds(sid * NROWS, NROWS)],
            agg_sh.at[pl.ds(sid * NROWS, NROWS)],
        )
        pltpu.sync_copy(s_hbm.at[t], s_v)
        pltpu.sync_copy(r_hbm.at[t], r_v)
        plsc.subcore_barrier()

        def issue_in(j, b):
            pltpu.async_copy(base_hbm.at[t, pl.ds(j * CH, CH)], b_v.at[b], sem_b)
            pltpu.async_copy(S_hbm.at[s_v.at[j]], sr_v.at[b], sem_g)
            pltpu.async_copy(R_hbm.at[r_v.at[j]], rr_v.at[b], sem_g)

        def wait_in(j, b):
            pltpu.make_async_copy(
                base_hbm.at[t, pl.ds(j * CH, CH)], b_v.at[b], sem_b).wait()
            pltpu.make_async_copy(S_hbm.at[s_v.at[j]], sr_v.at[b], sem_g).wait()
            pltpu.make_async_copy(R_hbm.at[r_v.at[j]], rr_v.at[b], sem_g).wait()

        def drain_store(j, b):
            if write_e:
                pltpu.make_async_copy(
                    e_v.at[b], e_out.at[t, pl.ds(j * CH, CH)], sem_sl).wait()
            pltpu.make_async_copy(
                e_v.at[b], agg_sh.at[r_v.at[j]], sem_si).wait()

        def compute(j, b):
            @pl.loop(0, CH, unroll=8)
            def _row(i):
                for h in range(2):
                    sl = pl.ds(h * 16, 16)
                    e_v[b, i, sl] = jnp.maximum(
                        b_v[b, i, sl] + sr_v[b, i, sl] + rr_v[b, i, sl], 0.0
                    )

        def issue_store(j, b):
            if write_e:
                pltpu.async_copy(
                    e_v.at[b], e_out.at[t, pl.ds(j * CH, CH)], sem_sl)
            pltpu.async_copy(e_v.at[b], agg_sh.at[r_v.at[j]], sem_si, add=True)

        issue_in(0, 0)

        @pl.loop(0, NCH // 2)
        def _pair(jj):
            for par in range(2):
                j = 2 * jj + par
                nb = 1 - par
                issue_in(j + 1, nb)
                wait_in(j, par)

                @pl.when(j >= 2)
                def _():
                    drain_store(j, par)

                compute(j, par)
                issue_store(j, par)

        jl = NCH - 1
        wait_in(jl, 0)
        drain_store(jl - 2, 0)
        compute(jl, 0)
        issue_store(jl, 0)
        drain_store(jl - 1, 1)
        drain_store(jl, 0)

        plsc.subcore_barrier()

        @pl.when(c == 0)
        def _():
            pltpu.sync_copy(
                agg_sh.at[pl.ds(sid * NROWS, NROWS)],
                agg_out0.at[pl.ds(sid * NROWS, NROWS)],
            )

        @pl.when(c == 1)
        def _():
            pltpu.sync_copy(
                agg_sh.at[pl.ds(sid * NROWS, NROWS)],
                agg_out1.at[pl.ds(sid * NROWS, NROWS)],
            )

    return pl.kernel(
        body,
        out_type=tuple(out_type),
        mesh=mesh,
        scratch_types=scratch,
        compiler_params=pltpu.CompilerParams(use_tc_tiling_on_sc=False),
    )


_sc_block_we = _make_sc_block(True)
_sc_block_agg = _make_sc_block(False)


NPK = NP // 4


def _mm(a, w):
    return jnp.dot(a, w, preferred_element_type=jnp.float32)


def _stage(groups, grid):
    arrays = []
    specs = []
    out_specs, out_shape = [], []
    plans = []

    def add(arr, spec):
        arrays.append(arr)
        specs.append(spec)
        return len(arrays) - 1

    for g in groups:
        if "t4" in g:
            ea, W, bias = g["t4"]
            K = ea.shape[1]
            colsb = E4 // grid
            panel_ids = [
                add(ea, pl.BlockSpec((colsb, K),
                                     lambda i, c=c: (c * grid + i, 0)))
                for c in range(4)
            ]
            w_id = add(W, pl.BlockSpec(W.shape, lambda i: (0, 0)))
            b = bias.reshape(1, -1)
            b_id = add(b, pl.BlockSpec(b.shape, lambda i: (0, 0)))
            out_specs.append(pl.BlockSpec((colsb, 128), lambda i: (i, 0)))
            out_shape.append(jax.ShapeDtypeStruct((E4, 128), jnp.float32))
            plans.append(("t4", panel_ids, w_id, b_id))
        else:
            rows = g["ins"][0].shape[0] if g["ins"] else NPK
            br = (E4 if rows == E4 else NPK) // grid
            in_ids = [
                add(a, pl.BlockSpec((br, a.shape[1]), lambda i: (i, 0)))
                for a in g["ins"]
            ]
            outs = []
            for terms, bias, relu in g["outs"]:
                t_ids = [
                    (kind, idx,
                     add(W, pl.BlockSpec(W.shape, lambda i: (0, 0))))
                    for kind, idx, W in terms
                ]
                b = bias.reshape(1, -1)
                b_id = add(b, pl.BlockSpec(b.shape, lambda i: (0, 0)))
                out_specs.append(pl.BlockSpec((br, b.shape[1]),
                                              lambda i: (i, 0)))
                out_shape.append(
                    jax.ShapeDtypeStruct((br * grid, b.shape[1]),
                                         jnp.float32))
                outs.append((t_ids, b_id, relu))
            plans.append(("gen", in_ids, outs))

    n_in = len(arrays)

    def body(*refs):
        in_refs = refs[:n_in]
        o_refs = refs[n_in:]
        oi = 0
        for plan in plans:
            if plan[0] == "t4":
                _, panel_ids, w_id, b_id = plan
                w = in_refs[w_id][...]
                vals = [_mm(in_refs[p][...], w) for p in panel_ids]
                o_refs[oi][...] = jnp.concatenate(vals, axis=1) \
                    + in_refs[b_id][...]
                oi += 1
            else:
                _, in_ids, outs = plan
                outvals = []
                for t_ids, b_id, relu in outs:
                    acc = in_refs[b_id][...]
                    for kind, idx, w_id in t_ids:
                        op = (in_refs[in_ids[idx]][...] if kind == "in"
                              else outvals[idx])
                        acc = acc + _mm(op, in_refs[w_id][...])
                    val = jnp.maximum(acc, 0.0) if relu else acc
                    outvals.append(val)
                    o_refs[oi][...] = val
                    oi += 1

    return pl.pallas_call(
        body,
        grid=(grid,),
        in_specs=specs,
        out_specs=out_specs,
        out_shape=out_shape,
        compiler_params=pltpu.CompilerParams(
            fuse_transposed_lhs_in_matmul=True),
    )(*arrays)


def _blockdiag(W, k):
    din, dout = W.shape
    out = jnp.zeros((k * din, k * dout), jnp.float32)
    for i in range(k):
        out = out.at[i * din:(i + 1) * din, i * dout:(i + 1) * dout].set(W)
    return out


def kernel(x, edge_attr, edge_index,
           enc_We, enc_be, enc_Wn, enc_bn,
           proc_We, proc_be, proc_Wn, proc_bn,
           dec_We, dec_be, dec_Wn, dec_bn):
    s3 = edge_index[0].reshape(4, E4).T.reshape(NW, NCH, CH)
    r3 = edge_index[1].reshape(4, E4).T.reshape(NW, NCH, CH)
    zeros = jnp.zeros((NP, L), jnp.float32)

    We_e = enc_We[:DE]
    We_s = enc_We[DE:DE + DF]
    We_r = enc_We[DE + DF:]
    W_ce = proc_We[0 * L:1 * L]
    W_ee = proc_We[1 * L:2 * L]
    W_scx = proc_We[2 * L:3 * L]
    W_sex = proc_We[3 * L:4 * L]
    W_rcx = proc_We[4 * L:5 * L]
    W_rex = proc_We[5 * L:6 * L]
    Wn_cx = proc_Wn[0 * L:1 * L]
    Wn_ex = proc_Wn[1 * L:2 * L]
    Wn_agg = proc_Wn[2 * L:3 * L]

    bd = functools.partial(_blockdiag, k=4)
    be4 = lambda b: jnp.tile(b, 4)

    z128 = jnp.zeros((128,), jnp.float32)

    xp = jnp.concatenate([x, jnp.zeros((NP - N, DF), jnp.float32)])
    x4 = xp.reshape(NPK, 4 * DF)
    pk = lambda a: a.reshape(NPK, 128)
    tb = lambda a: a.reshape(NP, L)

    (base1,) = _stage([
        {"t4": (edge_attr, We_e, be4(enc_be))},
    ], 25)
    S1, R1 = _stage([
        {"ins": [x4],
         "outs": [
             ([("in", 0, bd(We_s))], z128, False),
             ([("in", 0, bd(We_r))], z128, False),
         ]},
    ], 10)

    he3, a1c0, a1c1 = _sc_block_we(s3, r3, base1.reshape(NW, EW, L),
                                   tb(S1), tb(R1), zeros)
    he4 = he3.reshape(E4, 128)

    base2, hx, S2, R2 = _stage([
        {"ins": [he4],
         "outs": [([("in", 0, bd(W_ce + W_ee))], be4(proc_be), False)]},
        {"ins": [x4, pk(a1c0), pk(a1c1)],
         "outs": [
             ([("in", 0, bd(enc_Wn[:DF])), ("in", 1, bd(enc_Wn[DF:])),
               ("in", 2, bd(enc_Wn[DF:]))], be4(enc_bn), True),
             ([("out", 0, bd(W_scx + W_sex))], z128, False),
             ([("out", 0, bd(W_rcx + W_rex))], z128, False),
         ]},
    ], 20)

    ce13, a2c0, a2c1 = _sc_block_we(s3, r3, base2.reshape(NW, EW, L),
                                    tb(S2), tb(R2), zeros)
    ce14 = ce13.reshape(E4, 128)

    base3, cx1, S3, R3 = _stage([
        {"ins": [ce14, he4],
         "outs": [([("in", 0, bd(W_ce)), ("in", 1, bd(W_ee))],
                   be4(proc_be), False)]},
        {"ins": [hx, pk(a2c0), pk(a2c1)],
         "outs": [
             ([("in", 0, bd(Wn_cx + Wn_ex)), ("in", 1, bd(Wn_agg)),
               ("in", 2, bd(Wn_agg))], be4(proc_bn), True),
             ([("out", 0, bd(W_scx)), ("in", 0, bd(W_sex))], z128, False),
             ([("out", 0, bd(W_rcx)), ("in", 0, bd(W_rex))], z128, False),
         ]},
    ], 20)

    ce23, a3c0, a3c1 = _sc_block_we(s3, r3, base3.reshape(NW, EW, L),
                                    tb(S3), tb(R3), zeros)
    ce24 = ce23.reshape(E4, 128)

    base4, cx2, S4, R4 = _stage([
        {"ins": [ce24],
         "outs": [([("in", 0, bd(dec_We[:L]))], be4(dec_be), False)]},
        {"ins": [cx1, hx, pk(a3c0), pk(a3c1)],
         "outs": [
             ([("in", 0, bd(Wn_cx)), ("in", 1, bd(Wn_ex)),
               ("in", 2, bd(Wn_agg)), ("in", 3, bd(Wn_agg))],
              be4(proc_bn), True),
             ([("out", 0, bd(dec_We[L:2 * L]))], z128, False),
             ([("out", 0, bd(dec_We[2 * L:]))], z128, False),
         ]},
    ], 20)

    a4c0, a4c1 = _sc_block_agg(s3, r3, base4.reshape(NW, EW, L),
                               tb(S4), tb(R4), zeros)

    (outp,) = _stage([
        {"ins": [cx2, pk(a4c0), pk(a4c1)],
         "outs": [
             ([("in", 0, bd(dec_Wn[:L])), ("in", 1, bd(dec_Wn[L:])),
               ("in", 2, bd(dec_Wn[L:]))], be4(dec_bn), True),
         ]},
    ], 10)
    return outp.reshape(NP, L)[:N]


# speedup vs baseline: 1.4133x; 1.1930x over previous
import functools

import jax
import jax.numpy as jnp
from jax import lax
from jax.experimental import pallas as pl
from jax.experimental.pallas import tpu as pltpu
from jax.experimental.pallas import tpu_sc as plsc

N = 10000
E = 320000
DF = 128
DE = 16
L = 32

NC = 2
NS = 16
NW = NC * NS
EW = E // NW
CH = 80
NCH = EW // CH
NP = 10240
NROWS = NP // NS

E4 = E // 4


RW = 2512


def _make_sc_block(write_e: bool, permute: bool = False):
    mesh = plsc.VectorSubcoreMesh(
        core_axis_name="c", subcore_axis_name="s", num_cores=NC, num_subcores=NS
    )
    out_type = []
    if permute:
        out_type.append(jax.ShapeDtypeStruct((NW, NCH, CH), jnp.int32))
        out_type.append(jax.ShapeDtypeStruct((NW, NCH, CH), jnp.int32))
    if write_e:
        out_type.append(jax.ShapeDtypeStruct((NW, EW, L), jnp.float32))
    out_type.append(jax.ShapeDtypeStruct((NP, L), jnp.float32))
    out_type.append(jax.ShapeDtypeStruct((NP, L), jnp.float32))

    scratch = [
        pltpu.VMEM((NCH, CH), jnp.int32),
        pltpu.VMEM((NCH, CH), jnp.int32),
        pltpu.VMEM((2, CH, L), jnp.float32),
        pltpu.VMEM((2, CH, L), jnp.float32),
        pltpu.VMEM((2, CH, L), jnp.float32),
        pltpu.VMEM((2, CH, L), jnp.float32),
        pltpu.SemaphoreType.DMA,
        pltpu.SemaphoreType.DMA,
        pltpu.SemaphoreType.DMA,
        pltpu.SemaphoreType.DMA,
        pltpu.VMEM_SHARED((NP, L), jnp.float32),
    ]
    if permute:
        scratch += [
            pltpu.VMEM((4, RW), jnp.int32),
            pltpu.VMEM((4, RW), jnp.int32),
        ]

    def body(s_hbm, r_hbm, base_hbm, S_hbm, R_hbm, z_hbm, *rest):
        if permute:
            s_out, r_out = rest[0], rest[1]
            rest = rest[2:]
        if write_e:
            e_out, agg_out0, agg_out1 = rest[0], rest[1], rest[2]
            scr = rest[3:]
        else:
            agg_out0, agg_out1 = rest[0], rest[1]
            scr = rest[2:]
        (s_v, r_v, b_v, sr_v, rr_v, e_v,
         sem_b, sem_g, sem_sl, sem_si, agg_sh) = scr[:11]

        c = lax.axis_index("c")
        sid = lax.axis_index("s")
        t = c * NS + sid

        pltpu.sync_copy(
            z_hbm.at[pl.ds(sid * NROWS, NROWS)],
            agg_sh.at[pl.ds(sid * NROWS, NROWS)],
        )
        if permute:
            s_raw, r_raw = scr[11], scr[12]
            aoff = t * (EW // 4)
            a = (aoff // 16) * 16
            off = aoff - a
            pltpu.sync_copy(s_hbm.at[:, pl.ds(a, RW)], s_raw)
            pltpu.sync_copy(r_hbm.at[:, pl.ds(a, RW)], r_raw)
            iota = lax.iota(jnp.int32, 16)
            row_c = lax.rem(iota, 4)
            col_c = lax.div(iota, 4)

            @pl.loop(0, NCH)
            def _rj(j):
                for k in range(CH // 16):
                    ic = col_c + (off + j * (CH // 4) + k * 4)
                    s_v[j, pl.ds(k * 16, 16)] = plsc.load_gather(
                        s_raw, [row_c, ic])
                    r_v[j, pl.ds(k * 16, 16)] = plsc.load_gather(
                        r_raw, [row_c, ic])

            pltpu.sync_copy(s_v, s_out.at[t])
            pltpu.sync_copy(r_v, r_out.at[t])
        else:
            pltpu.sync_copy(s_hbm.at[t], s_v)
            pltpu.sync_copy(r_hbm.at[t], r_v)
        plsc.subcore_barrier()

        def issue_in(j, b):
            pltpu.async_copy(base_hbm.at[t, pl.ds(j * CH, CH)], b_v.at[b], sem_b)
            pltpu.async_copy(S_hbm.at[s_v.at[j]], sr_v.at[b], sem_g)
            pltpu.async_copy(R_hbm.at[r_v.at[j]], rr_v.at[b], sem_g)

        def wait_in(j, b):
            pltpu.make_async_copy(
                base_hbm.at[t, pl.ds(j * CH, CH)], b_v.at[b], sem_b).wait()
            pltpu.make_async_copy(S_hbm.at[s_v.at[j]], sr_v.at[b], sem_g).wait()
            pltpu.make_async_copy(R_hbm.at[r_v.at[j]], rr_v.at[b], sem_g).wait()

        def drain_store(j, b):
            if write_e:
                pltpu.make_async_copy(
                    e_v.at[b], e_out.at[t, pl.ds(j * CH, CH)], sem_sl).wait()
            pltpu.make_async_copy(
                e_v.at[b], agg_sh.at[r_v.at[j]], sem_si).wait()

        def compute(j, b):
            @pl.loop(0, CH, unroll=8)
            def _row(i):
                for h in range(2):
                    sl = pl.ds(h * 16, 16)
                    e_v[b, i, sl] = jnp.maximum(
                        b_v[b, i, sl] + sr_v[b, i, sl] + rr_v[b, i, sl], 0.0
                    )

        def issue_store(j, b):
            if write_e:
                pltpu.async_copy(
                    e_v.at[b], e_out.at[t, pl.ds(j * CH, CH)], sem_sl)
            pltpu.async_copy(e_v.at[b], agg_sh.at[r_v.at[j]], sem_si, add=True)

        issue_in(0, 0)

        @pl.loop(0, NCH // 2)
        def _pair(jj):
            for par in range(2):
                j = 2 * jj + par
                nb = 1 - par
                issue_in(j + 1, nb)
                wait_in(j, par)

                @pl.when(j >= 2)
                def _():
                    drain_store(j, par)

                compute(j, par)
                issue_store(j, par)

        jl = NCH - 1
        wait_in(jl, 0)
        drain_store(jl - 2, 0)
        compute(jl, 0)
        issue_store(jl, 0)
        drain_store(jl - 1, 1)
        drain_store(jl, 0)

        plsc.subcore_barrier()

        @pl.when(c == 0)
        def _():
            pltpu.sync_copy(
                agg_sh.at[pl.ds(sid * NROWS, NROWS)],
                agg_out0.at[pl.ds(sid * NROWS, NROWS)],
            )

        @pl.when(c == 1)
        def _():
            pltpu.sync_copy(
                agg_sh.at[pl.ds(sid * NROWS, NROWS)],
                agg_out1.at[pl.ds(sid * NROWS, NROWS)],
            )

    return pl.kernel(
        body,
        out_type=tuple(out_type),
        mesh=mesh,
        scratch_types=scratch,
        compiler_params=pltpu.CompilerParams(
            use_tc_tiling_on_sc=False,
            needs_layout_passes=False if permute else None,
        ),
    )


_sc_block_first = _make_sc_block(True, permute=True)
_sc_block_we = _make_sc_block(True)
_sc_block_agg = _make_sc_block(False)


NPK = NP // 4


def _mm(a, w):
    return jnp.dot(a, w, preferred_element_type=jnp.float32)


def _stage(groups, grid):
    arrays = []
    specs = []
    out_specs, out_shape = [], []
    plans = []

    def add(arr, spec):
        arrays.append(arr)
        specs.append(spec)
        return len(arrays) - 1

    for g in groups:
        if "t4" in g:
            eaT, W, bias = g["t4"]
            K = eaT.shape[0]
            colsb = E4 // grid
            panel_ids = [
                add(eaT, pl.BlockSpec((K, colsb),
                                      lambda i, c=c: (0, c * grid + i)))
                for c in range(4)
            ]
            w_id = add(W, pl.BlockSpec(W.shape, lambda i: (0, 0)))
            b = bias.reshape(1, -1)
            b_id = add(b, pl.BlockSpec(b.shape, lambda i: (0, 0)))
            out_specs.append(pl.BlockSpec((colsb, 128), lambda i: (i, 0)))
            out_shape.append(jax.ShapeDtypeStruct((E4, 128), jnp.float32))
            plans.append(("t4", panel_ids, w_id, b_id))
        else:
            rows = g["ins"][0].shape[0] if g["ins"] else NPK
            br = (E4 if rows == E4 else NPK) // grid
            in_ids = [
                add(a, pl.BlockSpec((br, a.shape[1]), lambda i: (i, 0)))
                for a in g["ins"]
            ]
            outs = []
            for terms, bias, relu in g["outs"]:
                t_ids = [
                    (kind, idx,
                     add(W, pl.BlockSpec(W.shape, lambda i: (0, 0))))
                    for kind, idx, W in terms
                ]
                b = bias.reshape(1, -1)
                b_id = add(b, pl.BlockSpec(b.shape, lambda i: (0, 0)))
                out_specs.append(pl.BlockSpec((br, b.shape[1]),
                                              lambda i: (i, 0)))
                out_shape.append(
                    jax.ShapeDtypeStruct((br * grid, b.shape[1]),
                                         jnp.float32))
                outs.append((t_ids, b_id, relu))
            plans.append(("gen", in_ids, outs))

    n_in = len(arrays)

    def body(*refs):
        in_refs = refs[:n_in]
        o_refs = refs[n_in:]
        oi = 0
        for plan in plans:
            if plan[0] == "t4":
                _, panel_ids, w_id, b_id = plan
                w = in_refs[w_id][...]
                vals = [
                    lax.dot_general(
                        in_refs[p][...], w, (((0,), (0,)), ((), ())),
                        preferred_element_type=jnp.float32)
                    for p in panel_ids
                ]
                o_refs[oi][...] = jnp.concatenate(vals, axis=1) \
                    + in_refs[b_id][...]
                oi += 1
            else:
                _, in_ids, outs = plan
                outvals = []
                for t_ids, b_id, relu in outs:
                    acc = in_refs[b_id][...]
                    for kind, idx, w_id in t_ids:
                        op = (in_refs[in_ids[idx]][...] if kind == "in"
                              else outvals[idx])
                        acc = acc + _mm(op, in_refs[w_id][...])
                    val = jnp.maximum(acc, 0.0) if relu else acc
                    outvals.append(val)
                    o_refs[oi][...] = val
                    oi += 1

    return pl.pallas_call(
        body,
        grid=(grid,),
        in_specs=specs,
        out_specs=out_specs,
        out_shape=out_shape,
        compiler_params=pltpu.CompilerParams(
            fuse_transposed_lhs_in_matmul=True),
    )(*arrays)


def _blockdiag(W, k):
    din, dout = W.shape
    out = jnp.zeros((k * din, k * dout), jnp.float32)
    for i in range(k):
        out = out.at[i * din:(i + 1) * din, i * dout:(i + 1) * dout].set(W)
    return out


def kernel(x, edge_attr, edge_index,
           enc_We, enc_be, enc_Wn, enc_bn,
           proc_We, proc_be, proc_Wn, proc_bn,
           dec_We, dec_be, dec_Wn, dec_bn):
    s4 = edge_index[0].reshape(4, E4)
    r4 = edge_index[1].reshape(4, E4)
    zeros = jnp.zeros((NP, L), jnp.float32)

    We_e = enc_We[:DE]
    We_s = enc_We[DE:DE + DF]
    We_r = enc_We[DE + DF:]
    W_ce = proc_We[0 * L:1 * L]
    W_ee = proc_We[1 * L:2 * L]
    W_scx = proc_We[2 * L:3 * L]
    W_sex = proc_We[3 * L:4 * L]
    W_rcx = proc_We[4 * L:5 * L]
    W_rex = proc_We[5 * L:6 * L]
    Wn_cx = proc_Wn[0 * L:1 * L]
    Wn_ex = proc_Wn[1 * L:2 * L]
    Wn_agg = proc_Wn[2 * L:3 * L]

    bd = functools.partial(_blockdiag, k=4)
    be4 = lambda b: jnp.tile(b, 4)

    z128 = jnp.zeros((128,), jnp.float32)

    xp = jnp.concatenate([x, jnp.zeros((NP - N, DF), jnp.float32)])
    x4 = xp.reshape(NPK, 4 * DF)
    pk = lambda a: a.reshape(NPK, 128)
    tb = lambda a: a.reshape(NP, L)

    (base1,) = _stage([
        {"t4": (edge_attr.T, We_e, be4(enc_be))},
    ], 25)
    S1, R1 = _stage([
        {"ins": [x4],
         "outs": [
             ([("in", 0, bd(We_s))], z128, False),
             ([("in", 0, bd(We_r))], z128, False),
         ]},
    ], 10)

    s3, r3, he3, a1c0, a1c1 = _sc_block_first(
        s4, r4, base1.reshape(NW, EW, L), tb(S1), tb(R1), zeros)
    he4 = he3.reshape(E4, 128)

    base2, hx, S2, R2 = _stage([
        {"ins": [he4],
         "outs": [([("in", 0, bd(W_ce + W_ee))], be4(proc_be), False)]},
        {"ins": [x4, pk(a1c0), pk(a1c1)],
         "outs": [
             ([("in", 0, bd(enc_Wn[:DF])), ("in", 1, bd(enc_Wn[DF:])),
               ("in", 2, bd(enc_Wn[DF:]))], be4(enc_bn), True),
             ([("out", 0, bd(W_scx + W_sex))], z128, False),
             ([("out", 0, bd(W_rcx + W_rex))], z128, False),
         ]},
    ], 20)

    ce13, a2c0, a2c1 = _sc_block_we(s3, r3, base2.reshape(NW, EW, L),
                                    tb(S2), tb(R2), zeros)
    ce14 = ce13.reshape(E4, 128)

    base3, cx1, S3, R3 = _stage([
        {"ins": [ce14, he4],
         "outs": [([("in", 0, bd(W_ce)), ("in", 1, bd(W_ee))],
                   be4(proc_be), False)]},
        {"ins": [hx, pk(a2c0), pk(a2c1)],
         "outs": [
             ([("in", 0, bd(Wn_cx + Wn_ex)), ("in", 1, bd(Wn_agg)),
               ("in", 2, bd(Wn_agg))], be4(proc_bn), True),
             ([("out", 0, bd(W_scx)), ("in", 0, bd(W_sex))], z128, False),
             ([("out", 0, bd(W_rcx)), ("in", 0, bd(W_rex))], z128, False),
         ]},
    ], 20)

    ce23, a3c0, a3c1 = _sc_block_we(s3, r3, base3.reshape(NW, EW, L),
                                    tb(S3), tb(R3), zeros)
    ce24 = ce23.reshape(E4, 128)

    base4, cx2, S4, R4 = _stage([
        {"ins": [ce24],
         "outs": [([("in", 0, bd(dec_We[:L]))], be4(dec_be), False)]},
        {"ins": [cx1, hx, pk(a3c0), pk(a3c1)],
         "outs": [
             ([("in", 0, bd(Wn_cx)), ("in", 1, bd(Wn_ex)),
               ("in", 2, bd(Wn_agg)), ("in", 3, bd(Wn_agg))],
              be4(proc_bn), True),
             ([("out", 0, bd(dec_We[L:2 * L]))], z128, False),
             ([("out", 0, bd(dec_We[2 * L:]))], z128, False),
         ]},
    ], 20)

    a4c0, a4c1 = _sc_block_agg(s3, r3, base4.reshape(NW, EW, L),
                               tb(S4), tb(R4), zeros)

    (outp,) = _stage([
        {"ins": [cx2, pk(a4c0), pk(a4c1)],
         "outs": [
             ([("in", 0, bd(dec_Wn[:L])), ("in", 1, bd(dec_Wn[L:])),
               ("in", 2, bd(dec_Wn[L:]))], be4(dec_bn), True),
         ]},
    ], 10)
    return outp.reshape(NP, L)[:N]
